# baseline (speedup 1.0000x reference)
import jax
import jax.numpy as jnp
from jax.experimental import pallas as pl

BATCH = 512
DIM_CODES = 16
BOOK_SIZE = 8192
EMB_DIM = 32
W_DIM = DIM_CODES * EMB_DIM

BB = 128


def _vq_body(xt_ref, cb_ref, oh_ref, cwe_ref):
    xd = xt_ref[0]
    cb = cb_ref[0]
    x2 = jnp.sum(xd * xd, axis=1, keepdims=True)
    c2 = jnp.sum(cb * cb, axis=1)[None, :]
    cross = jax.lax.dot_general(
        xd, cb, (((1,), (1,)), ((), ())),
        preferred_element_type=jnp.float32)
    dist = x2 + c2 - 2.0 * cross
    mind = jnp.min(dist, axis=1, keepdims=True)
    iota = jax.lax.broadcasted_iota(jnp.int32, dist.shape, 1)
    idx = jnp.min(jnp.where(dist == mind, iota, BOOK_SIZE),
                  axis=1, keepdims=True)
    oh = (iota == idx).astype(jnp.float32)
    oh_ref[...] = oh
    cwe_ref[0] = jax.lax.dot_general(
        oh, cb, (((1,), (0,)), ((), ())),
        preferred_element_type=jnp.float32)


def kernel(x, codebook):
    batch = x.shape[0]
    nb = batch // BB
    xt = x.reshape(batch, DIM_CODES, EMB_DIM).transpose(1, 0, 2)

    oh_flat, cwe_t = pl.pallas_call(
        _vq_body,
        grid=(DIM_CODES, nb),
        in_specs=[
            pl.BlockSpec((1, BB, EMB_DIM), lambda d, b: (d, b, 0)),
            pl.BlockSpec((1, BOOK_SIZE, EMB_DIM), lambda d, b: (d, 0, 0)),
        ],
        out_specs=[
            pl.BlockSpec((BB, BOOK_SIZE), lambda d, b: (b, d)),
            pl.BlockSpec((1, BB, EMB_DIM), lambda d, b: (d, b, 0)),
        ],
        out_shape=[
            jax.ShapeDtypeStruct((batch, DIM_CODES * BOOK_SIZE), jnp.float32),
            jax.ShapeDtypeStruct((DIM_CODES, batch, EMB_DIM), jnp.float32),
        ],
    )(xt, codebook)

    one_hot = oh_flat.reshape(batch, DIM_CODES, BOOK_SIZE)
    cw_e = cwe_t.transpose(1, 0, 2).reshape(batch, W_DIM)
    cw = x + jax.lax.stop_gradient(cw_e - x)
    return cw, cw_e, one_hot


# speedup vs baseline: 1.4934x; 1.4934x over previous
import jax
import jax.numpy as jnp
from jax.experimental import pallas as pl
from jax.experimental.pallas import tpu as pltpu

BATCH = 512
DIM_CODES = 16
BOOK_SIZE = 8192
EMB_DIM = 32
W_DIM = DIM_CODES * EMB_DIM
EA = EMB_DIM + 16

BB1 = 128
BB2 = 32


def _argmin_body(xt_ref, cbt_ref, idxm_ref, cwet_ref):
    iota = jax.lax.broadcasted_iota(jnp.int32, (BB1, BOOK_SIZE), 1)

    def dstep(d, carry):
        cbd = cbt_ref[d]
        c2 = jnp.sum(cbd * cbd, axis=0)[None, :]
        xd = xt_ref[d]
        cross = jax.lax.dot_general(
            xd, cbd, (((1,), (0,)), ((), ())),
            preferred_element_type=jnp.float32)
        score = c2 - 2.0 * cross
        m = jnp.min(score, axis=1, keepdims=True)
        idx = jnp.min(jnp.where(score == m, iota, BOOK_SIZE),
                      axis=1, keepdims=True)
        oh_b = (iota == idx).astype(jnp.float32)
        cweT = jax.lax.dot_general(
            cbd, oh_b.T, (((1,), (0,)), ((), ())),
            preferred_element_type=jnp.float32)
        cwet_ref[d] = cweT.T
        idxm_ref[d] = idx.T
        return carry

    jax.lax.fori_loop(0, DIM_CODES, dstep, 0)


def _onehot_body(idx_ref, oh_ref):
    idxT = idx_ref[...].T
    iota2 = jax.lax.broadcasted_iota(jnp.int32, (DIM_CODES, BOOK_SIZE), 1)
    for b in range(BB2):
        col = idxT[:, b:b + 1]
        oh_ref[b] = (iota2 == col).astype(jnp.float32)


def kernel(x, codebook):
    batch = x.shape[0]
    cbt = codebook.transpose(0, 2, 1)
    xt = x.reshape(batch, DIM_CODES, EMB_DIM).transpose(1, 0, 2)

    idxm, cwet = pl.pallas_call(
        _argmin_body,
        grid=(batch // BB1,),
        in_specs=[
            pl.BlockSpec((DIM_CODES, BB1, EMB_DIM), lambda b: (0, b, 0)),
            pl.BlockSpec((DIM_CODES, EMB_DIM, BOOK_SIZE), lambda b: (0, 0, 0)),
        ],
        out_specs=[
            pl.BlockSpec((DIM_CODES, 1, BB1), lambda b: (0, 0, b)),
            pl.BlockSpec((DIM_CODES, BB1, EMB_DIM), lambda b: (0, b, 0)),
        ],
        out_shape=[
            jax.ShapeDtypeStruct((DIM_CODES, 1, batch), jnp.int32),
            jax.ShapeDtypeStruct((DIM_CODES, batch, EMB_DIM), jnp.float32),
        ],
    )(xt, cbt)

    idxf = idxm.reshape(DIM_CODES, batch).T
    cwe = cwet.transpose(1, 0, 2).reshape(batch, W_DIM)
    cw = x + jax.lax.stop_gradient(cwe - x)

    oh = pl.pallas_call(
        _onehot_body,
        grid=(batch // BB2,),
        in_specs=[pl.BlockSpec((BB2, DIM_CODES), lambda b: (b, 0))],
        out_specs=pl.BlockSpec((BB2, DIM_CODES, BOOK_SIZE), lambda b: (b, 0, 0)),
        out_shape=jax.ShapeDtypeStruct((batch, DIM_CODES, BOOK_SIZE),
                                       jnp.float32),
    )(idxf)

    return cw, cwe, oh


# speedup vs baseline: 1.7306x; 1.1588x over previous
import functools

import jax
import jax.numpy as jnp
from jax import lax
from jax.experimental import pallas as pl
from jax.experimental.pallas import tpu as pltpu
from jax.experimental.pallas import tpu_sc as plsc

BATCH = 512
DIM_CODES = 16
BOOK_SIZE = 8192
EMB_DIM = 32
W_DIM = DIM_CODES * EMB_DIM

BB1 = 128
BB2 = 32
DHALF = DIM_CODES // 2


def _argmin_body(xt_ref, cbt_ref, idxm_ref):
    iota = jax.lax.broadcasted_iota(jnp.int32, (BB1, BOOK_SIZE), 1)

    def one_d(d):
        cbd = cbt_ref[d]
        c2 = jnp.sum(cbd * cbd, axis=0)[None, :]
        xd = xt_ref[d]
        cross = jax.lax.dot_general(
            xd, cbd, (((1,), (0,)), ((), ())),
            preferred_element_type=jnp.float32)
        score = c2 - 2.0 * cross
        m = jnp.min(score, axis=1, keepdims=True)
        idx = jnp.min(jnp.where(score == m, iota, BOOK_SIZE),
                      axis=1, keepdims=True)
        idxm_ref[d] = idx.T

    def dstep(d, carry):
        one_d(d)
        one_d(d + DHALF)
        return carry

    jax.lax.fori_loop(0, DHALF, dstep, 0)


def _onehot_body(idx_ref, oh_ref):
    idxT = idx_ref[...].T
    iota2 = jax.lax.broadcasted_iota(jnp.int32, (DIM_CODES, BOOK_SIZE), 1)
    for b in range(BB2):
        col = idxT[:, b:b + 1]
        oh_ref[b] = (iota2 == col).astype(jnp.float32)


def _sc_gather(table_ref, idx_ref, out_ref, idx_v, rows_v, sem):
    wid = lax.axis_index("s") * 2 + lax.axis_index("c")
    d = wid % DIM_CODES
    h = wid // DIM_CODES
    bper = BATCH // 2
    b0 = h * bper
    pltpu.sync_copy(idx_ref.at[d, pl.ds(b0, bper)], idx_v)
    off = jnp.int32(d * BOOK_SIZE)
    for i in range(bper // 16):
        sl = pl.ds(i * 16, 16)
        idx_v[sl] = idx_v[sl] + off
    for j in range(2):
        sl = pl.ds(j * 128, 128)
        pltpu.async_copy(table_ref.at[idx_v.at[sl]], rows_v.at[sl], sem).wait()
    pltpu.sync_copy(rows_v, out_ref.at[pl.ds(b0, bper), d])


def kernel(x, codebook):
    batch = x.shape[0]
    cbt = codebook.transpose(0, 2, 1)
    xt = x.reshape(batch, DIM_CODES, EMB_DIM).transpose(1, 0, 2)

    idxm = pl.pallas_call(
        _argmin_body,
        grid=(batch // BB1,),
        in_specs=[
            pl.BlockSpec((DIM_CODES, BB1, EMB_DIM), lambda b: (0, b, 0)),
            pl.BlockSpec((DIM_CODES, EMB_DIM, BOOK_SIZE), lambda b: (0, 0, 0)),
        ],
        out_specs=pl.BlockSpec((DIM_CODES, 1, BB1), lambda b: (0, 0, b)),
        out_shape=jax.ShapeDtypeStruct((DIM_CODES, 1, batch), jnp.int32),
    )(xt, cbt)

    idxdm = idxm.reshape(DIM_CODES, batch)
    table = codebook.reshape(DIM_CODES * BOOK_SIZE, EMB_DIM)

    sc_gather = functools.partial(
        pl.kernel,
        out_type=jax.ShapeDtypeStruct((batch, DIM_CODES, EMB_DIM),
                                      jnp.float32),
        mesh=plsc.VectorSubcoreMesh(core_axis_name="c", subcore_axis_name="s"),
        compiler_params=pltpu.CompilerParams(use_tc_tiling_on_sc=False),
        scratch_types=[
            pltpu.VMEM((batch // 2,), jnp.int32),
            pltpu.VMEM((batch // 2, EMB_DIM), jnp.float32),
            pltpu.SemaphoreType.DMA,
        ],
    )(_sc_gather)

    cwe = sc_gather(table, idxdm).reshape(batch, W_DIM)
    cw = x + jax.lax.stop_gradient(cwe - x)

    idxf = idxdm.T
    oh = pl.pallas_call(
        _onehot_body,
        grid=(batch // BB2,),
        in_specs=[pl.BlockSpec((BB2, DIM_CODES), lambda b: (b, 0))],
        out_specs=pl.BlockSpec((BB2, DIM_CODES, BOOK_SIZE), lambda b: (b, 0, 0)),
        out_shape=jax.ShapeDtypeStruct((batch, DIM_CODES, BOOK_SIZE),
                                       jnp.float32),
    )(idxf)

    return cw, cwe, oh


# speedup vs baseline: 1.8403x; 1.0634x over previous
import functools

import jax
import jax.numpy as jnp
from jax import lax
from jax.experimental import pallas as pl
from jax.experimental.pallas import tpu as pltpu
from jax.experimental.pallas import tpu_sc as plsc

BATCH = 512
DIM_CODES = 16
BOOK_SIZE = 8192
EMB_DIM = 32
W_DIM = DIM_CODES * EMB_DIM

BB1 = 128
BB2 = 32
DQ = DIM_CODES // 4


def _argmin_body(xt_ref, cbt_ref, idxm_ref):
    iota = jax.lax.broadcasted_iota(jnp.int32, (BB1, BOOK_SIZE), 1)

    def one_d(d):
        cbd = cbt_ref[d]
        c2 = jnp.sum(cbd * cbd, axis=0)[None, :]
        xd = xt_ref[d]
        cross = jax.lax.dot_general(
            xd, cbd, (((1,), (0,)), ((), ())),
            preferred_element_type=jnp.float32)
        score = c2 - 2.0 * cross
        m = jnp.min(score, axis=1, keepdims=True)
        idx = jnp.min(jnp.where(score == m, iota, BOOK_SIZE),
                      axis=1, keepdims=True)
        idxm_ref[d] = idx.T

    def dstep(d, carry):
        one_d(d)
        one_d(d + DQ)
        one_d(d + 2 * DQ)
        one_d(d + 3 * DQ)
        return carry

    jax.lax.fori_loop(0, DQ, dstep, 0)


def _onehot_body(idx_ref, oh_ref):
    idxT = idx_ref[...].T
    iota2 = jax.lax.broadcasted_iota(jnp.int32, (DIM_CODES, BOOK_SIZE), 1)
    for b in range(BB2):
        col = idxT[:, b:b + 1]
        oh_ref[b] = (iota2 == col).astype(jnp.float32)


def _sc_gather(cb_ref, idx_ref, out_ref, idx_v, rows_v, sem):
    wid = lax.axis_index("s") * 2 + lax.axis_index("c")
    d = wid % DIM_CODES
    h = wid // DIM_CODES
    bper = BATCH // 2
    b0 = h * bper
    pltpu.sync_copy(idx_ref.at[d, pl.ds(b0, bper)], idx_v)

    chunk = 16

    def cstep(c, carry):
        base = c * chunk
        vec = idx_v[pl.ds(base, chunk)]
        copies = []
        for j in range(chunk):
            copies.append(pltpu.async_copy(
                cb_ref.at[d, vec[j]], rows_v.at[base + j], sem))
        for cp in copies:
            cp.wait()
        return carry

    jax.lax.fori_loop(0, bper // chunk, cstep, 0)
    pltpu.sync_copy(rows_v, out_ref.at[pl.ds(b0, bper), d])


def kernel(x, codebook):
    batch = x.shape[0]
    cbt = codebook.transpose(0, 2, 1)
    xt = x.reshape(batch, DIM_CODES, EMB_DIM).transpose(1, 0, 2)

    idxm = pl.pallas_call(
        _argmin_body,
        grid=(batch // BB1,),
        in_specs=[
            pl.BlockSpec((DIM_CODES, BB1, EMB_DIM), lambda b: (0, b, 0)),
            pl.BlockSpec((DIM_CODES, EMB_DIM, BOOK_SIZE), lambda b: (0, 0, 0)),
        ],
        out_specs=pl.BlockSpec((DIM_CODES, 1, BB1), lambda b: (0, 0, b)),
        out_shape=jax.ShapeDtypeStruct((DIM_CODES, 1, batch), jnp.int32),
    )(xt, cbt)

    idxdm = idxm.reshape(DIM_CODES, batch)

    sc_gather = functools.partial(
        pl.kernel,
        out_type=jax.ShapeDtypeStruct((batch, DIM_CODES, EMB_DIM),
                                      jnp.float32),
        mesh=plsc.VectorSubcoreMesh(core_axis_name="c", subcore_axis_name="s"),
        scratch_types=[
            pltpu.VMEM((batch // 2,), jnp.int32),
            pltpu.VMEM((batch // 2, EMB_DIM), jnp.float32),
            pltpu.SemaphoreType.DMA,
        ],
    )(_sc_gather)

    cwe = sc_gather(codebook, idxdm).reshape(batch, W_DIM)
    cw = x + jax.lax.stop_gradient(cwe - x)

    idxf = idxdm.T
    oh = pl.pallas_call(
        _onehot_body,
        grid=(batch // BB2,),
        in_specs=[pl.BlockSpec((BB2, DIM_CODES), lambda b: (b, 0))],
        out_specs=pl.BlockSpec((BB2, DIM_CODES, BOOK_SIZE), lambda b: (b, 0, 0)),
        out_shape=jax.ShapeDtypeStruct((batch, DIM_CODES, BOOK_SIZE),
                                       jnp.float32),
    )(idxf)

    return cw, cwe, oh


# speedup vs baseline: 2.0173x; 1.0962x over previous
import functools

import jax
import jax.numpy as jnp
from jax import lax
from jax.experimental import pallas as pl
from jax.experimental.pallas import tpu as pltpu
from jax.experimental.pallas import tpu_sc as plsc

BATCH = 512
DIM_CODES = 16
BOOK_SIZE = 8192
EMB_DIM = 32
W_DIM = DIM_CODES * EMB_DIM

BB = 32
DQ = DIM_CODES // 4


def _vq_body(xt2_ref, cbt_ref, oh_ref, idx3_ref, c2_ref, ixa_ref):
    @pl.when(pl.program_id(0) == 0)
    def _():
        def cinit(d, carry):
            cbd = cbt_ref[d]
            c2_ref[d] = jnp.sum(cbd * cbd, axis=0, keepdims=True)
            return carry
        jax.lax.fori_loop(0, DIM_CODES, cinit, 0)

    iota = jax.lax.broadcasted_iota(jnp.int32, (BB, BOOK_SIZE), 1)

    def one_d(d):
        cross2 = jax.lax.dot_general(
            xt2_ref[d], cbt_ref[d], (((1,), (0,)), ((), ())),
            preferred_element_type=jnp.float32)
        score = c2_ref[d] + cross2
        m = jnp.min(score, axis=1, keepdims=True)
        idx = jnp.min(jnp.where(score == m, iota, BOOK_SIZE),
                      axis=1, keepdims=True)
        ixa_ref[d] = idx.T

    def dstep(d, carry):
        one_d(d)
        one_d(d + DQ)
        one_d(d + 2 * DQ)
        one_d(d + 3 * DQ)
        return carry

    jax.lax.fori_loop(0, DQ, dstep, 0)

    ixa = ixa_ref[...]
    idx3_ref[...] = ixa[:, 0, :].T
    iota2 = jax.lax.broadcasted_iota(jnp.int32, (DIM_CODES, BOOK_SIZE), 1)
    for b in range(BB):
        col = ixa[:, 0, b:b + 1]
        oh_ref[b] = (iota2 == col).astype(jnp.float32)


def _sc_gather(cb_ref, idx_ref, out_ref, idx_v, rows_v, sem):
    wid = lax.axis_index("s") * 2 + lax.axis_index("c")
    d = wid % DIM_CODES
    h = wid // DIM_CODES
    bper = BATCH // 2
    b0 = h * bper
    pltpu.sync_copy(idx_ref.at[d, pl.ds(b0, bper)], idx_v)

    chunk = 16

    def cstep(c, carry):
        base = c * chunk
        vec = idx_v[pl.ds(base, chunk)]
        copies = []
        for j in range(chunk):
            copies.append(pltpu.async_copy(
                cb_ref.at[d, vec[j]], rows_v.at[base + j], sem))
        for cp in copies:
            cp.wait()
        return carry

    jax.lax.fori_loop(0, bper // chunk, cstep, 0)
    pltpu.sync_copy(rows_v, out_ref.at[pl.ds(b0, bper), d])


def kernel(x, codebook):
    batch = x.shape[0]
    cbt = codebook.transpose(0, 2, 1)
    xt2 = (-2.0 * x).reshape(batch, DIM_CODES, EMB_DIM).transpose(1, 0, 2)

    oh, idx3 = pl.pallas_call(
        _vq_body,
        grid=(batch // BB,),
        in_specs=[
            pl.BlockSpec((DIM_CODES, BB, EMB_DIM), lambda b: (0, b, 0)),
            pl.BlockSpec((DIM_CODES, EMB_DIM, BOOK_SIZE), lambda b: (0, 0, 0)),
        ],
        out_specs=[
            pl.BlockSpec((BB, DIM_CODES, BOOK_SIZE), lambda b: (b, 0, 0)),
            pl.BlockSpec((BB, DIM_CODES), lambda b: (b, 0)),
        ],
        out_shape=[
            jax.ShapeDtypeStruct((batch, DIM_CODES, BOOK_SIZE), jnp.float32),
            jax.ShapeDtypeStruct((batch, DIM_CODES), jnp.int32),
        ],
        scratch_shapes=[
            pltpu.VMEM((DIM_CODES, 1, BOOK_SIZE), jnp.float32),
            pltpu.VMEM((DIM_CODES, 1, BB), jnp.int32),
        ],
    )(xt2, cbt)


    sc_gather = functools.partial(
        pl.kernel,
        out_type=jax.ShapeDtypeStruct((batch, DIM_CODES, EMB_DIM),
                                      jnp.float32),
        mesh=plsc.VectorSubcoreMesh(core_axis_name="c", subcore_axis_name="s"),
        scratch_types=[
            pltpu.VMEM((batch // 2,), jnp.int32),
            pltpu.VMEM((batch // 2, EMB_DIM), jnp.float32),
            pltpu.SemaphoreType.DMA,
        ],
    )(_sc_gather)

    cwe = sc_gather(codebook, idx3.T).reshape(batch, W_DIM)
    cw = x + jax.lax.stop_gradient(cwe - x)
    return cw, cwe, oh


# speedup vs baseline: 2.0924x; 1.0373x over previous
import functools

import jax
import jax.numpy as jnp
from jax import lax
from jax.experimental import pallas as pl
from jax.experimental.pallas import tpu as pltpu
from jax.experimental.pallas import tpu_sc as plsc

BATCH = 512
DIM_CODES = 16
BOOK_SIZE = 8192
EMB_DIM = 32
W_DIM = DIM_CODES * EMB_DIM

BB = 32
DQ = DIM_CODES // 4


def _vq_body(xt2_ref, cbt_ref, oh_ref, idx3_ref, c2_ref, ixa_ref):
    @pl.when(pl.program_id(0) == 0)
    def _():
        def cinit(d, carry):
            cbd = cbt_ref[d]
            c2_ref[d] = jnp.sum(cbd * cbd, axis=0, keepdims=True)
            return carry
        jax.lax.fori_loop(0, DIM_CODES, cinit, 0)

    iota = jax.lax.broadcasted_iota(jnp.int32, (BB, BOOK_SIZE), 1)

    def one_d(d):
        cross2 = jax.lax.dot_general(
            xt2_ref[d], cbt_ref[d], (((1,), (0,)), ((), ())),
            preferred_element_type=jnp.float32)
        score = c2_ref[d] + cross2
        m = jnp.min(score, axis=1, keepdims=True)
        idx = jnp.min(jnp.where(score == m, iota, BOOK_SIZE),
                      axis=1, keepdims=True)
        ixa_ref[d] = idx.T

    for dd in range(DQ):
        one_d(dd)
        one_d(dd + DQ)
        one_d(dd + 2 * DQ)
        one_d(dd + 3 * DQ)

    ixa = ixa_ref[...]
    idx3_ref[...] = ixa[:, 0, :].T
    iota2 = jax.lax.broadcasted_iota(jnp.int32, (DIM_CODES, BOOK_SIZE), 1)
    for b in range(BB):
        col = ixa[:, 0, b:b + 1]
        oh_ref[b] = (iota2 == col).astype(jnp.float32)


def _sc_gather(cb_ref, idx_ref, out_ref, idx_v, rows_v, sem):
    wid = lax.axis_index("s") * 2 + lax.axis_index("c")
    d = wid % DIM_CODES
    h = wid // DIM_CODES
    bper = BATCH // 2
    b0 = h * bper
    pltpu.sync_copy(idx_ref.at[d, pl.ds(b0, bper)], idx_v)

    chunk = 16

    def cstep(c, carry):
        base = c * chunk
        vec = idx_v[pl.ds(base, chunk)]
        copies = []
        for j in range(chunk):
            copies.append(pltpu.async_copy(
                cb_ref.at[d, vec[j]], rows_v.at[base + j], sem))
        for cp in copies:
            cp.wait()
        return carry

    jax.lax.fori_loop(0, bper // chunk, cstep, 0)
    pltpu.sync_copy(rows_v, out_ref.at[pl.ds(b0, bper), d])


def kernel(x, codebook):
    batch = x.shape[0]
    cbt = codebook.transpose(0, 2, 1)
    xt2 = (-2.0 * x).reshape(batch, DIM_CODES, EMB_DIM).transpose(1, 0, 2)

    oh, idx3 = pl.pallas_call(
        _vq_body,
        grid=(batch // BB,),
        in_specs=[
            pl.BlockSpec((DIM_CODES, BB, EMB_DIM), lambda b: (0, b, 0)),
            pl.BlockSpec((DIM_CODES, EMB_DIM, BOOK_SIZE), lambda b: (0, 0, 0)),
        ],
        out_specs=[
            pl.BlockSpec((BB, DIM_CODES, BOOK_SIZE), lambda b: (b, 0, 0)),
            pl.BlockSpec((BB, DIM_CODES), lambda b: (b, 0)),
        ],
        out_shape=[
            jax.ShapeDtypeStruct((batch, DIM_CODES, BOOK_SIZE), jnp.float32),
            jax.ShapeDtypeStruct((batch, DIM_CODES), jnp.int32),
        ],
        scratch_shapes=[
            pltpu.VMEM((DIM_CODES, 1, BOOK_SIZE), jnp.float32),
            pltpu.VMEM((DIM_CODES, 1, BB), jnp.int32),
        ],
    )(xt2, cbt)


    sc_gather = functools.partial(
        pl.kernel,
        out_type=jax.ShapeDtypeStruct((batch, DIM_CODES, EMB_DIM),
                                      jnp.float32),
        mesh=plsc.VectorSubcoreMesh(core_axis_name="c", subcore_axis_name="s"),
        scratch_types=[
            pltpu.VMEM((batch // 2,), jnp.int32),
            pltpu.VMEM((batch // 2, EMB_DIM), jnp.float32),
            pltpu.SemaphoreType.DMA,
        ],
    )(_sc_gather)

    cwe = sc_gather(codebook, idx3.T).reshape(batch, W_DIM)
    cw = x + jax.lax.stop_gradient(cwe - x)
    return cw, cwe, oh


# speedup vs baseline: 2.2527x; 1.0766x over previous
import functools

import jax
import jax.numpy as jnp
from jax import lax
from jax.experimental import pallas as pl
from jax.experimental.pallas import tpu as pltpu
from jax.experimental.pallas import tpu_sc as plsc

BATCH = 512
DIM_CODES = 16
BOOK_SIZE = 8192
EMB_DIM = 32
W_DIM = DIM_CODES * EMB_DIM

BB = 32
DQ = DIM_CODES // 4


def _vq_body(xt2_ref, cbt_ref, oh_ref, idx3_ref, c2_ref, ixa_ref):
    @pl.when(pl.program_id(0) == 0)
    def _():
        def cinit(d, carry):
            cbd = cbt_ref[d]
            c2_ref[d] = jnp.sum(cbd * cbd, axis=0, keepdims=True)
            return carry
        jax.lax.fori_loop(0, DIM_CODES, cinit, 0)

    iota = jax.lax.broadcasted_iota(jnp.int32, (BB, BOOK_SIZE), 1)

    def one_d(d):
        cross2 = jax.lax.dot_general(
            xt2_ref[d], cbt_ref[d], (((1,), (0,)), ((), ())),
            preferred_element_type=jnp.float32)
        score = c2_ref[d] + cross2
        m = jnp.min(score, axis=1, keepdims=True)
        idx = jnp.min(jnp.where(score == m, iota, BOOK_SIZE),
                      axis=1, keepdims=True)
        ixa_ref[d] = idx.T

    for dd in range(DQ):
        one_d(dd)
        one_d(dd + DQ)
        one_d(dd + 2 * DQ)
        one_d(dd + 3 * DQ)

    ixa = ixa_ref[...]
    idx3_ref[...] = ixa[:, 0, :].T
    iota2 = jax.lax.broadcasted_iota(jnp.int32, (DIM_CODES, BOOK_SIZE), 1)
    for b in range(BB):
        col = ixa[:, 0, b:b + 1]
        oh_ref[b] = (iota2 == col).astype(jnp.float32)


def _sc_gather(cb_ref, x_ref, idx_ref, cwe_ref, cw_ref,
               idx_v, rows_v, xrows_v, cw_v, sem):
    wid = lax.axis_index("s") * 2 + lax.axis_index("c")
    bw = BATCH // 32
    r0 = wid * bw
    pltpu.sync_copy(idx_ref.at[pl.ds(r0, bw)], idx_v)
    xcp = pltpu.async_copy(x_ref.at[pl.ds(r0, bw)], xrows_v, sem)

    def jstep(j, carry):
        vec = idx_v[j]
        copies = []
        for dd in range(DIM_CODES):
            copies.append(pltpu.async_copy(
                cb_ref.at[dd, vec[dd]],
                rows_v.at[j, pl.ds(dd * EMB_DIM, EMB_DIM)], sem))
        for cp in copies:
            cp.wait()
        return carry

    jax.lax.fori_loop(0, bw, jstep, 0)
    xcp.wait()

    nch = W_DIM // 16

    def estep(j, carry):
        for c in range(nch):
            sl = pl.ds(c * 16, 16)
            vx = xrows_v[j, sl]
            vc = rows_v[j, sl]
            cw_v[j, sl] = vx + (vc - vx)
        return carry

    jax.lax.fori_loop(0, bw, estep, 0)
    pltpu.sync_copy(rows_v, cwe_ref.at[pl.ds(r0, bw)])
    pltpu.sync_copy(cw_v, cw_ref.at[pl.ds(r0, bw)])


def kernel(x, codebook):
    batch = x.shape[0]
    cbt = codebook.transpose(0, 2, 1)
    xt2 = (-2.0 * x).reshape(batch, DIM_CODES, EMB_DIM).transpose(1, 0, 2)

    oh, idx3 = pl.pallas_call(
        _vq_body,
        grid=(batch // BB,),
        in_specs=[
            pl.BlockSpec((DIM_CODES, BB, EMB_DIM), lambda b: (0, b, 0)),
            pl.BlockSpec((DIM_CODES, EMB_DIM, BOOK_SIZE), lambda b: (0, 0, 0)),
        ],
        out_specs=[
            pl.BlockSpec((BB, DIM_CODES, BOOK_SIZE), lambda b: (b, 0, 0)),
            pl.BlockSpec((BB, DIM_CODES), lambda b: (b, 0)),
        ],
        out_shape=[
            jax.ShapeDtypeStruct((batch, DIM_CODES, BOOK_SIZE), jnp.float32),
            jax.ShapeDtypeStruct((batch, DIM_CODES), jnp.int32),
        ],
        scratch_shapes=[
            pltpu.VMEM((DIM_CODES, 1, BOOK_SIZE), jnp.float32),
            pltpu.VMEM((DIM_CODES, 1, BB), jnp.int32),
        ],
    )(xt2, cbt)


    bw = batch // 32
    sc_gather = functools.partial(
        pl.kernel,
        out_type=[
            jax.ShapeDtypeStruct((batch, W_DIM), jnp.float32),
            jax.ShapeDtypeStruct((batch, W_DIM), jnp.float32),
        ],
        mesh=plsc.VectorSubcoreMesh(core_axis_name="c", subcore_axis_name="s"),
        scratch_types=[
            pltpu.VMEM((bw, DIM_CODES), jnp.int32),
            pltpu.VMEM((bw, W_DIM), jnp.float32),
            pltpu.VMEM((bw, W_DIM), jnp.float32),
            pltpu.VMEM((bw, W_DIM), jnp.float32),
            pltpu.SemaphoreType.DMA,
        ],
    )(_sc_gather)

    cwe, cw = sc_gather(codebook, x, idx3)
    return cw, cwe, oh
